# Initial kernel scaffold; baseline (speedup 1.0000x reference)
import jax
import jax.numpy as jnp
from jax.experimental import pallas as pl


def kernel(input_ids, note_type_ids, concentration_ids, position_ids, season_ids, emotion_ids, time_ids, token_table, note_table, conc_table, pos_table, season_table, emotion_table, time_table):
    raise NotImplementedError("write your pallas kernel here")


# speedup vs baseline: 5.4238x; 5.4238x over previous
import math

import jax
import jax.numpy as jnp
from jax import lax
from jax.experimental import pallas as pl
from jax.experimental.pallas import tpu as pltpu
from jax.experimental.pallas import tpu_sc as plsc

NC = 2
NS = 16
L = 16
NW = NC * NS

B = 1024
S = 200
D = 128
N = B * S
RPW = N // NW
BPW = B // NW
CH = 128
NCH = RPW // CH
NJ = D // L
SCALE = math.sqrt(D)


def _body(tok_ids, pos_ids, note_ids, conc_ids, sids, eids, tids,
          tok_tab, note_tab, conc_tab, pos_tab, sea_tab, emo_tab, tim_tab,
          out,
          ids_tok_v, ids_pos_v, ids_note_v, ids_conc_v,
          note_v, conc_v, sea_v, emo_v, tim_v, combo_v, bvec_v,
          sid_v, eid_v, tid_v, tok_buf, pos_buf, sem_t, sem_p):
    wid = lax.axis_index("s") * NC + lax.axis_index("c")
    row0 = wid * RPW
    c0 = wid * NCH
    bg0 = wid * BPW

    pltpu.sync_copy(tok_ids.at[pl.ds(c0, NCH)], ids_tok_v)
    pltpu.sync_copy(pos_ids.at[pl.ds(c0, NCH)], ids_pos_v)
    pltpu.sync_copy(note_ids.at[pl.ds(c0, NCH)], ids_note_v)
    pltpu.sync_copy(conc_ids.at[pl.ds(c0, NCH)], ids_conc_v)
    pltpu.sync_copy(note_tab, note_v)
    pltpu.sync_copy(conc_tab, conc_v)
    pltpu.sync_copy(sea_tab, sea_v)
    pltpu.sync_copy(emo_tab, emo_v)
    pltpu.sync_copy(tim_tab, tim_v)
    pltpu.sync_copy(sids.at[pl.ds(bg0, BPW)], sid_v)
    pltpu.sync_copy(eids.at[pl.ds(bg0, BPW)], eid_v)
    pltpu.sync_copy(tids.at[pl.ds(bg0, BPW)], tid_v)

    cols = [lax.iota(jnp.int32, L) + j * L for j in range(NJ)]

    def combo_step(c, carry):
        n = c // 20
        k = c - n * 20
        for j in range(NJ):
            combo_v[c, pl.ds(j * L, L)] = (
                note_v[n, pl.ds(j * L, L)] + conc_v[k, pl.ds(j * L, L)])
        return carry
    lax.fori_loop(0, 60, combo_step, 0)

    for g in range(BPW // L):
        sv = sid_v[pl.ds(g * L, L)]
        ev = eid_v[pl.ds(g * L, L)]
        tv = tid_v[pl.ds(g * L, L)]
        for r in range(L):
            for j in range(NJ):
                bvec_v[g * L + r, pl.ds(j * L, L)] = (
                    sea_v[sv[r], pl.ds(j * L, L)]
                    + emo_v[ev[r], pl.ds(j * L, L)]
                    + tim_v[tv[r], pl.ds(j * L, L)])

    def chunk_step(c, carry):
        pltpu.async_copy(tok_tab.at[ids_tok_v.at[c]], tok_buf, sem_t).wait()
        pltpu.async_copy(pos_tab.at[ids_pos_v.at[c]], pos_buf, sem_p).wait()

        r0 = c * CH
        bA = r0 // S
        mid = jnp.minimum((bA + 1) * S - r0, CH)
        bB = jnp.minimum(bA + 1, BPW - 1)
        bvA = [bvec_v[bA, pl.ds(j * L, L)] for j in range(NJ)]
        bvB = [bvec_v[bB, pl.ds(j * L, L)] for j in range(NJ)]

        def h_step(h, inner):
            nidv = ids_note_v[c, pl.ds(h * L, L)]
            cidv = ids_conc_v[c, pl.ds(h * L, L)]
            cxv = nidv * 20 + cidv
            base = h * L
            for r in range(L):
                i = base + r
                ridx = jnp.full((L,), cxv[r], jnp.int32)
                pred = i >= mid
                for j in range(NJ):
                    tv = tok_buf[i, pl.ds(j * L, L)]
                    pv = pos_buf[i, pl.ds(j * L, L)]
                    cv = plsc.load_gather(combo_v, [ridx, cols[j]])
                    bvj = jnp.where(pred, bvB[j], bvA[j])
                    tok_buf[i, pl.ds(j * L, L)] = (
                        tv * SCALE + pv + cv + bvj)
            return inner
        lax.fori_loop(0, CH // L, h_step, 0)

        pltpu.sync_copy(tok_buf, out.at[pl.ds(row0 + r0, CH)])
        return carry
    lax.fori_loop(0, NCH, chunk_step, 0)


def kernel(input_ids, note_type_ids, concentration_ids, position_ids,
           season_ids, emotion_ids, time_ids,
           token_table, note_table, conc_table, pos_table,
           season_table, emotion_table, time_table):
    tok_ids = input_ids.reshape(N // CH, CH).astype(jnp.int32)
    pos_ids2 = position_ids.reshape(N // CH, CH).astype(jnp.int32)
    note_ids2 = note_type_ids.reshape(N // CH, CH).astype(jnp.int32)
    conc_ids2 = concentration_ids.reshape(N // CH, CH).astype(jnp.int32)
    sids = season_ids.astype(jnp.int32)
    eids = emotion_ids.astype(jnp.int32)
    tids = time_ids.astype(jnp.int32)

    mesh = plsc.VectorSubcoreMesh(
        core_axis_name="c", subcore_axis_name="s",
        num_cores=NC, num_subcores=NS)
    run = pl.kernel(
        _body,
        out_type=jax.ShapeDtypeStruct((N, D), jnp.float32),
        mesh=mesh,
        compiler_params=pltpu.CompilerParams(
            use_tc_tiling_on_sc=False, needs_layout_passes=False),
        scratch_types=[
            pltpu.VMEM((NCH, CH), jnp.int32),
            pltpu.VMEM((NCH, CH), jnp.int32),
            pltpu.VMEM((NCH, CH), jnp.int32),
            pltpu.VMEM((NCH, CH), jnp.int32),
            pltpu.VMEM((3, D), jnp.float32),
            pltpu.VMEM((20, D), jnp.float32),
            pltpu.VMEM((4, D), jnp.float32),
            pltpu.VMEM((8, D), jnp.float32),
            pltpu.VMEM((4, D), jnp.float32),
            pltpu.VMEM((60, D), jnp.float32),
            pltpu.VMEM((BPW, D), jnp.float32),
            pltpu.VMEM((BPW,), jnp.int32),
            pltpu.VMEM((BPW,), jnp.int32),
            pltpu.VMEM((BPW,), jnp.int32),
            pltpu.VMEM((CH, D), jnp.float32),
            pltpu.VMEM((CH, D), jnp.float32),
            pltpu.SemaphoreType.DMA,
            pltpu.SemaphoreType.DMA,
        ],
    )
    out = run(tok_ids, pos_ids2, note_ids2, conc_ids2, sids, eids, tids,
              token_table, note_table, conc_table, pos_table,
              season_table, emotion_table, time_table)
    return out.reshape(B, S, D)
